# baseline (device time: 1129727 ns/iter reference)
import jax
import jax.numpy as jnp
from jax import lax
from jax.experimental import pallas as pl
from jax.experimental.pallas import tpu as pltpu

M = 16384
N = 2048
NCOL = N // 2
NCHUNK = 8
CH = M // NCHUNK


def kernel(x):
    x = x.astype(jnp.bfloat16)

    def body(x_ref, out_ref, sendbuf, copy_sem, load_sems, send_sems, recv_sems):
        my_x = lax.axis_index("x")
        my_y = lax.axis_index("y")
        partner_y = 1 - my_y

        barrier_sem = pltpu.get_barrier_semaphore()
        pl.semaphore_signal(
            barrier_sem, inc=1,
            device_id=(my_x, partner_y),
            device_id_type=pl.DeviceIdType.MESH,
        )
        pl.semaphore_wait(barrier_sem, 1)

        local_copy = pltpu.make_async_copy(
            x_ref.at[:, pl.ds(my_y * NCOL, NCOL)],
            out_ref.at[pl.ds(my_y * M, M), :],
            copy_sem,
        )
        local_copy.start()

        loads = []
        for k in range(NCHUNK):
            ld = pltpu.make_async_copy(
                x_ref.at[pl.ds(k * CH, CH), pl.ds(partner_y * NCOL, NCOL)],
                sendbuf.at[k],
                load_sems.at[k],
            )
            ld.start()
            loads.append(ld)

        rdmas = []
        for k in range(NCHUNK):
            loads[k].wait()
            rdma = pltpu.make_async_remote_copy(
                src_ref=sendbuf.at[k],
                dst_ref=out_ref.at[pl.ds(my_y * M + k * CH, CH), :],
                send_sem=send_sems.at[k],
                recv_sem=recv_sems.at[k],
                device_id=(my_x, partner_y),
                device_id_type=pl.DeviceIdType.MESH,
            )
            rdma.start()
            rdmas.append(rdma)

        for k in range(NCHUNK):
            rdmas[k].wait_send()
        for k in range(NCHUNK):
            rdmas[k].wait_recv()
        local_copy.wait()

    return pl.pallas_call(
        body,
        out_shape=jax.ShapeDtypeStruct((2 * M, NCOL), jnp.bfloat16),
        in_specs=[pl.BlockSpec(memory_space=pl.ANY)],
        out_specs=pl.BlockSpec(memory_space=pl.ANY),
        scratch_shapes=[
            pltpu.VMEM((NCHUNK, CH, NCOL), jnp.bfloat16),
            pltpu.SemaphoreType.DMA,
            pltpu.SemaphoreType.DMA((NCHUNK,)),
            pltpu.SemaphoreType.DMA((NCHUNK,)),
            pltpu.SemaphoreType.DMA((NCHUNK,)),
        ],
        compiler_params=pltpu.CompilerParams(collective_id=0),
    )(x)


# device time: 1127308 ns/iter; 1.0021x vs baseline; 1.0021x over previous
import jax
import jax.numpy as jnp
from jax import lax
from jax.experimental import pallas as pl
from jax.experimental.pallas import tpu as pltpu

M = 16384
N = 2048
NCOL = N // 2
NCHUNK = 8
CH = M // NCHUNK


def kernel(x):
    x = x.astype(jnp.bfloat16)

    def body(x_ref, out_ref, recvbuf, copy_sem, store_sems, send_sems,
             recv_sems):
        my_x = lax.axis_index("x")
        my_y = lax.axis_index("y")
        partner_y = 1 - my_y

        barrier_sem = pltpu.get_barrier_semaphore()
        pl.semaphore_signal(
            barrier_sem, inc=1,
            device_id=(my_x, partner_y),
            device_id_type=pl.DeviceIdType.MESH,
        )
        pl.semaphore_wait(barrier_sem, 1)

        local_copy = pltpu.make_async_copy(
            x_ref.at[:, pl.ds(my_y * NCOL, NCOL)],
            out_ref.at[pl.ds(my_y * M, M), :],
            copy_sem,
        )
        local_copy.start()

        rdmas = []
        for k in range(NCHUNK):
            rdma = pltpu.make_async_remote_copy(
                src_ref=x_ref.at[pl.ds(k * CH, CH), pl.ds(partner_y * NCOL, NCOL)],
                dst_ref=recvbuf.at[k],
                send_sem=send_sems.at[k],
                recv_sem=recv_sems.at[k],
                device_id=(my_x, partner_y),
                device_id_type=pl.DeviceIdType.MESH,
            )
            rdma.start()
            rdmas.append(rdma)

        stores = []
        for k in range(NCHUNK):
            rdmas[k].wait_recv()
            st = pltpu.make_async_copy(
                recvbuf.at[k],
                out_ref.at[pl.ds(partner_y * M + k * CH, CH), :],
                store_sems.at[k],
            )
            st.start()
            stores.append(st)

        for k in range(NCHUNK):
            rdmas[k].wait_send()
            stores[k].wait()
        local_copy.wait()

    return pl.pallas_call(
        body,
        out_shape=jax.ShapeDtypeStruct((2 * M, NCOL), jnp.bfloat16),
        in_specs=[pl.BlockSpec(memory_space=pl.ANY)],
        out_specs=pl.BlockSpec(memory_space=pl.ANY),
        scratch_shapes=[
            pltpu.VMEM((NCHUNK, CH, NCOL), jnp.bfloat16),
            pltpu.SemaphoreType.DMA,
            pltpu.SemaphoreType.DMA((NCHUNK,)),
            pltpu.SemaphoreType.DMA((NCHUNK,)),
            pltpu.SemaphoreType.DMA((NCHUNK,)),
        ],
        compiler_params=pltpu.CompilerParams(collective_id=0),
    )(x)


# device time: 473114 ns/iter; 2.3879x vs baseline; 2.3827x over previous
import jax
import jax.numpy as jnp
from jax import lax
from jax.experimental import pallas as pl
from jax.experimental.pallas import tpu as pltpu

M = 16384
N = 2048
NCOL = N // 2
NCHUNK = 8
CH = M // NCHUNK
NLOCAL = 4


def kernel(x):
    x = x.astype(jnp.bfloat16)

    def body(x_ref, out_ref, recvbuf, localbuf, local_ld_sems, local_st_sems,
             store_sems, send_sems, recv_sems):
        my_x = lax.axis_index("x")
        my_y = lax.axis_index("y")
        partner_y = 1 - my_y

        barrier_sem = pltpu.get_barrier_semaphore()
        pl.semaphore_signal(
            barrier_sem, inc=1,
            device_id=(my_x, partner_y),
            device_id_type=pl.DeviceIdType.MESH,
        )
        pl.semaphore_wait(barrier_sem, 1)

        rdmas = []
        for k in range(NCHUNK):
            rdma = pltpu.make_async_remote_copy(
                src_ref=x_ref.at[pl.ds(k * CH, CH), pl.ds(partner_y * NCOL, NCOL)],
                dst_ref=recvbuf.at[k],
                send_sem=send_sems.at[k],
                recv_sem=recv_sems.at[k],
                device_id=(my_x, partner_y),
                device_id_type=pl.DeviceIdType.MESH,
            )
            rdma.start()
            rdmas.append(rdma)

        local_stores = []
        for k in range(NCHUNK):
            slot = k % NLOCAL
            if k >= NLOCAL:
                local_stores[k - NLOCAL].wait()
            ld = pltpu.make_async_copy(
                x_ref.at[pl.ds(k * CH, CH), pl.ds(my_y * NCOL, NCOL)],
                localbuf.at[slot],
                local_ld_sems.at[slot],
            )
            ld.start()
            ld.wait()
            st = pltpu.make_async_copy(
                localbuf.at[slot],
                out_ref.at[pl.ds(my_y * M + k * CH, CH), :],
                local_st_sems.at[slot],
            )
            st.start()
            local_stores.append(st)

        stores = []
        for k in range(NCHUNK):
            rdmas[k].wait_recv()
            st = pltpu.make_async_copy(
                recvbuf.at[k],
                out_ref.at[pl.ds(partner_y * M + k * CH, CH), :],
                store_sems.at[k],
            )
            st.start()
            stores.append(st)

        for k in range(NCHUNK):
            rdmas[k].wait_send()
            stores[k].wait()
        for k in range(NCHUNK - NLOCAL, NCHUNK):
            local_stores[k].wait()

    return pl.pallas_call(
        body,
        out_shape=jax.ShapeDtypeStruct((2 * M, NCOL), jnp.bfloat16),
        in_specs=[pl.BlockSpec(memory_space=pl.ANY)],
        out_specs=pl.BlockSpec(memory_space=pl.ANY),
        scratch_shapes=[
            pltpu.VMEM((NCHUNK, CH, NCOL), jnp.bfloat16),
            pltpu.VMEM((NLOCAL, CH, NCOL), jnp.bfloat16),
            pltpu.SemaphoreType.DMA((NLOCAL,)),
            pltpu.SemaphoreType.DMA((NLOCAL,)),
            pltpu.SemaphoreType.DMA((NCHUNK,)),
            pltpu.SemaphoreType.DMA((NCHUNK,)),
            pltpu.SemaphoreType.DMA((NCHUNK,)),
        ],
        compiler_params=pltpu.CompilerParams(
            collective_id=0, vmem_limit_bytes=63 * 1024 * 1024
        ),
    )(x)


# device time: 322680 ns/iter; 3.5011x vs baseline; 1.4662x over previous
import jax
import jax.numpy as jnp
from jax import lax
from jax.experimental import pallas as pl
from jax.experimental.pallas import tpu as pltpu

M = 16384
N = 2048
NCOL = N // 2
H = M // 2

NC = 8
CH = H // NC

NLC = 8
CHL = M // NLC
NLOCAL = 4


def kernel(x):
    x = x.astype(jnp.bfloat16)

    def body(x_ref, out_ref, ybuf, xbuf, localbuf,
             y_send_sems, y_recv_sems, x_send_sems, x_recv_sems,
             ystore_sems, xstore_sems, local_ld_sems, local_st_sems):
        my_x = lax.axis_index("x")
        my_y = lax.axis_index("y")
        partner_y = 1 - my_y
        partner_x = 1 - my_x

        barrier_sem = pltpu.get_barrier_semaphore()
        pl.semaphore_signal(
            barrier_sem, inc=1,
            device_id=(my_x, partner_y),
            device_id_type=pl.DeviceIdType.MESH,
        )
        pl.semaphore_signal(
            barrier_sem, inc=1,
            device_id=(partner_x, my_y),
            device_id_type=pl.DeviceIdType.MESH,
        )
        pl.semaphore_wait(barrier_sem, 2)

        y_rdmas = []
        for k in range(NC):
            rdma = pltpu.make_async_remote_copy(
                src_ref=x_ref.at[pl.ds(my_x * H + k * CH, CH),
                                 pl.ds(partner_y * NCOL, NCOL)],
                dst_ref=ybuf.at[k],
                send_sem=y_send_sems.at[k],
                recv_sem=y_recv_sems.at[k],
                device_id=(my_x, partner_y),
                device_id_type=pl.DeviceIdType.MESH,
            )
            rdma.start()
            y_rdmas.append(rdma)

        local_stores = []
        for k in range(NLC):
            slot = k % NLOCAL
            if k >= NLOCAL:
                local_stores[k - NLOCAL].wait()
            ld = pltpu.make_async_copy(
                x_ref.at[pl.ds(k * CHL, CHL), pl.ds(my_y * NCOL, NCOL)],
                localbuf.at[slot],
                local_ld_sems.at[slot],
            )
            ld.start()
            ld.wait()
            st = pltpu.make_async_copy(
                localbuf.at[slot],
                out_ref.at[pl.ds(my_y * M + k * CHL, CHL), :],
                local_st_sems.at[slot],
            )
            st.start()
            local_stores.append(st)

        x_rdmas = []
        ystores = []
        for k in range(NC):
            y_rdmas[k].wait_recv()
            fwd = pltpu.make_async_remote_copy(
                src_ref=ybuf.at[k],
                dst_ref=xbuf.at[k],
                send_sem=x_send_sems.at[k],
                recv_sem=x_recv_sems.at[k],
                device_id=(partner_x, my_y),
                device_id_type=pl.DeviceIdType.MESH,
            )
            fwd.start()
            x_rdmas.append(fwd)
            st = pltpu.make_async_copy(
                ybuf.at[k],
                out_ref.at[pl.ds(partner_y * M + my_x * H + k * CH, CH), :],
                ystore_sems.at[k],
            )
            st.start()
            ystores.append(st)

        xstores = []
        for k in range(NC):
            x_rdmas[k].wait_recv()
            st = pltpu.make_async_copy(
                xbuf.at[k],
                out_ref.at[pl.ds(partner_y * M + partner_x * H + k * CH, CH), :],
                xstore_sems.at[k],
            )
            st.start()
            xstores.append(st)

        for k in range(NC):
            y_rdmas[k].wait_send()
            x_rdmas[k].wait_send()
            ystores[k].wait()
            xstores[k].wait()
        for k in range(NLC - NLOCAL, NLC):
            local_stores[k].wait()

    return pl.pallas_call(
        body,
        out_shape=jax.ShapeDtypeStruct((2 * M, NCOL), jnp.bfloat16),
        in_specs=[pl.BlockSpec(memory_space=pl.ANY)],
        out_specs=pl.BlockSpec(memory_space=pl.ANY),
        scratch_shapes=[
            pltpu.VMEM((NC, CH, NCOL), jnp.bfloat16),
            pltpu.VMEM((NC, CH, NCOL), jnp.bfloat16),
            pltpu.VMEM((NLOCAL, CHL, NCOL), jnp.bfloat16),
            pltpu.SemaphoreType.DMA((NC,)),
            pltpu.SemaphoreType.DMA((NC,)),
            pltpu.SemaphoreType.DMA((NC,)),
            pltpu.SemaphoreType.DMA((NC,)),
            pltpu.SemaphoreType.DMA((NC,)),
            pltpu.SemaphoreType.DMA((NC,)),
            pltpu.SemaphoreType.DMA((NLOCAL,)),
            pltpu.SemaphoreType.DMA((NLOCAL,)),
        ],
        compiler_params=pltpu.CompilerParams(
            collective_id=0, vmem_limit_bytes=63 * 1024 * 1024
        ),
    )(x)


# device time: 254530 ns/iter; 4.4385x vs baseline; 1.2677x over previous
import jax
import jax.numpy as jnp
from jax import lax
from jax.experimental import pallas as pl
from jax.experimental.pallas import tpu as pltpu

M = 16384
N = 2048
NCOL = N // 2
H = M // 2

NC = 8
CH = H // NC
NSEND = 4

NLC = 16
CHL = M // NLC


def kernel(x):
    def body(x_ref, out_ref, ystage, sendbuf, ybuf, xbuf, lstage, lbuf,
             yld_sems, lld_sems, y_send_sems, y_recv_sems,
             x_send_sems, x_recv_sems, ystore_sems, xstore_sems, lst_sems):
        my_x = lax.axis_index("x")
        my_y = lax.axis_index("y")
        partner_y = 1 - my_y
        partner_x = 1 - my_x

        barrier_sem = pltpu.get_barrier_semaphore()
        pl.semaphore_signal(
            barrier_sem, inc=1,
            device_id=(my_x, partner_y),
            device_id_type=pl.DeviceIdType.MESH,
        )
        pl.semaphore_signal(
            barrier_sem, inc=1,
            device_id=(partner_x, my_y),
            device_id_type=pl.DeviceIdType.MESH,
        )
        pl.semaphore_wait(barrier_sem, 2)

        def yload(k):
            ld = pltpu.make_async_copy(
                x_ref.at[pl.ds(my_x * H + k * CH, CH),
                         pl.ds(partner_y * NCOL, NCOL)],
                ystage.at[k % 2],
                yld_sems.at[k % 2],
            )
            ld.start()
            return ld

        def lload(j):
            ld = pltpu.make_async_copy(
                x_ref.at[pl.ds(j * CHL, CHL), pl.ds(my_y * NCOL, NCOL)],
                lstage.at[j % 2],
                lld_sems.at[j % 2],
            )
            ld.start()
            return ld

        ylds = {0: yload(0)}
        llds = {0: lload(0)}
        y_rdmas = []
        x_rdmas = []
        ystores = []
        lstores = []

        def drain(k):
            y_rdmas[k].wait_recv()
            fwd = pltpu.make_async_remote_copy(
                src_ref=ybuf.at[k],
                dst_ref=xbuf.at[k],
                send_sem=x_send_sems.at[k],
                recv_sem=x_recv_sems.at[k],
                device_id=(partner_x, my_y),
                device_id_type=pl.DeviceIdType.MESH,
            )
            fwd.start()
            x_rdmas.append(fwd)
            st = pltpu.make_async_copy(
                ybuf.at[k],
                out_ref.at[pl.ds(partner_y * M + my_x * H + k * CH, CH), :],
                ystore_sems.at[k],
            )
            st.start()
            ystores.append(st)

        for k in range(NC):
            if k + 1 < NC:
                ylds[k + 1] = yload(k + 1)
            ylds[k].wait()
            if k >= NSEND:
                y_rdmas[k - NSEND].wait_send()
            sendbuf[k % NSEND] = ystage[k % 2].astype(jnp.bfloat16)
            rdma = pltpu.make_async_remote_copy(
                src_ref=sendbuf.at[k % NSEND],
                dst_ref=ybuf.at[k],
                send_sem=y_send_sems.at[k],
                recv_sem=y_recv_sems.at[k],
                device_id=(my_x, partner_y),
                device_id_type=pl.DeviceIdType.MESH,
            )
            rdma.start()
            y_rdmas.append(rdma)

            for j in (2 * k, 2 * k + 1):
                if j + 1 < NLC:
                    llds[j + 1] = lload(j + 1)
                llds[j].wait()
                if j >= 2:
                    lstores[j - 2].wait()
                lbuf[j % 2] = lstage[j % 2].astype(jnp.bfloat16)
                st = pltpu.make_async_copy(
                    lbuf.at[j % 2],
                    out_ref.at[pl.ds(my_y * M + j * CHL, CHL), :],
                    lst_sems.at[j % 2],
                )
                st.start()
                lstores.append(st)

            if k >= 1:
                drain(k - 1)

        drain(NC - 1)

        xstores = []
        for k in range(NC):
            x_rdmas[k].wait_recv()
            st = pltpu.make_async_copy(
                xbuf.at[k],
                out_ref.at[pl.ds(partner_y * M + partner_x * H + k * CH, CH), :],
                xstore_sems.at[k],
            )
            st.start()
            xstores.append(st)

        for k in range(NC - NSEND, NC):
            y_rdmas[k].wait_send()
        for k in range(NC):
            x_rdmas[k].wait_send()
            ystores[k].wait()
            xstores[k].wait()
        for j in (NLC - 2, NLC - 1):
            lstores[j].wait()

    return pl.pallas_call(
        body,
        out_shape=jax.ShapeDtypeStruct((2 * M, NCOL), jnp.bfloat16),
        in_specs=[pl.BlockSpec(memory_space=pl.ANY)],
        out_specs=pl.BlockSpec(memory_space=pl.ANY),
        scratch_shapes=[
            pltpu.VMEM((2, CH, NCOL), jnp.float32),
            pltpu.VMEM((NSEND, CH, NCOL), jnp.bfloat16),
            pltpu.VMEM((NC, CH, NCOL), jnp.bfloat16),
            pltpu.VMEM((NC, CH, NCOL), jnp.bfloat16),
            pltpu.VMEM((2, CHL, NCOL), jnp.float32),
            pltpu.VMEM((2, CHL, NCOL), jnp.bfloat16),
            pltpu.SemaphoreType.DMA((2,)),
            pltpu.SemaphoreType.DMA((2,)),
            pltpu.SemaphoreType.DMA((NC,)),
            pltpu.SemaphoreType.DMA((NC,)),
            pltpu.SemaphoreType.DMA((NC,)),
            pltpu.SemaphoreType.DMA((NC,)),
            pltpu.SemaphoreType.DMA((NC,)),
            pltpu.SemaphoreType.DMA((NC,)),
            pltpu.SemaphoreType.DMA((2,)),
        ],
        compiler_params=pltpu.CompilerParams(
            collective_id=0, vmem_limit_bytes=63 * 1024 * 1024
        ),
    )(x)


# device time: 242689 ns/iter; 4.6550x vs baseline; 1.0488x over previous
import jax
import jax.numpy as jnp
from jax import lax
from jax.experimental import pallas as pl
from jax.experimental.pallas import tpu as pltpu

M = 16384
N = 2048
NCOL = N // 2
H = M // 2

NC = 16
CH = H // NC
NSEND = 8

NLC = 16
CHL = M // NLC


def kernel(x):
    def body(x_ref, out_ref, ystage, sendbuf, ybuf, xbuf, lstage, lbuf,
             yld_sems, lld_sems, y_send_sems, y_recv_sems,
             x_send_sems, x_recv_sems, ystore_sems, xstore_sems, lst_sems):
        my_x = lax.axis_index("x")
        my_y = lax.axis_index("y")
        partner_y = 1 - my_y
        partner_x = 1 - my_x

        barrier_sem = pltpu.get_barrier_semaphore()
        pl.semaphore_signal(
            barrier_sem, inc=1,
            device_id=(my_x, partner_y),
            device_id_type=pl.DeviceIdType.MESH,
        )
        pl.semaphore_signal(
            barrier_sem, inc=1,
            device_id=(partner_x, my_y),
            device_id_type=pl.DeviceIdType.MESH,
        )
        pl.semaphore_wait(barrier_sem, 2)

        def yload(k):
            ld = pltpu.make_async_copy(
                x_ref.at[pl.ds(my_x * H + k * CH, CH),
                         pl.ds(partner_y * NCOL, NCOL)],
                ystage.at[k % 2],
                yld_sems.at[k % 2],
            )
            ld.start()
            return ld

        def lload(j):
            ld = pltpu.make_async_copy(
                x_ref.at[pl.ds(j * CHL, CHL), pl.ds(my_y * NCOL, NCOL)],
                lstage.at[j % 2],
                lld_sems.at[j % 2],
            )
            ld.start()
            return ld

        ylds = {0: yload(0)}
        llds = {0: lload(0)}
        y_rdmas = []
        x_rdmas = []
        ystores = []
        lstores = []

        def drain(k):
            y_rdmas[k].wait_recv()
            fwd = pltpu.make_async_remote_copy(
                src_ref=ybuf.at[k],
                dst_ref=xbuf.at[k],
                send_sem=x_send_sems.at[k],
                recv_sem=x_recv_sems.at[k],
                device_id=(partner_x, my_y),
                device_id_type=pl.DeviceIdType.MESH,
            )
            fwd.start()
            x_rdmas.append(fwd)
            st = pltpu.make_async_copy(
                ybuf.at[k],
                out_ref.at[pl.ds(partner_y * M + my_x * H + k * CH, CH), :],
                ystore_sems.at[k],
            )
            st.start()
            ystores.append(st)

        for k in range(NC):
            if k + 1 < NC:
                ylds[k + 1] = yload(k + 1)
            ylds[k].wait()
            if k >= NSEND:
                y_rdmas[k - NSEND].wait_send()
            sendbuf[k % NSEND] = ystage[k % 2].astype(jnp.bfloat16)
            rdma = pltpu.make_async_remote_copy(
                src_ref=sendbuf.at[k % NSEND],
                dst_ref=ybuf.at[k],
                send_sem=y_send_sems.at[k],
                recv_sem=y_recv_sems.at[k],
                device_id=(my_x, partner_y),
                device_id_type=pl.DeviceIdType.MESH,
            )
            rdma.start()
            y_rdmas.append(rdma)

            for j in range((NLC // NC) * k, (NLC // NC) * (k + 1)):
                if j + 1 < NLC:
                    llds[j + 1] = lload(j + 1)
                llds[j].wait()
                if j >= 2:
                    lstores[j - 2].wait()
                lbuf[j % 2] = lstage[j % 2].astype(jnp.bfloat16)
                st = pltpu.make_async_copy(
                    lbuf.at[j % 2],
                    out_ref.at[pl.ds(my_y * M + j * CHL, CHL), :],
                    lst_sems.at[j % 2],
                )
                st.start()
                lstores.append(st)

            if k >= 1:
                drain(k - 1)

        drain(NC - 1)

        xstores = []
        for k in range(NC):
            x_rdmas[k].wait_recv()
            st = pltpu.make_async_copy(
                xbuf.at[k],
                out_ref.at[pl.ds(partner_y * M + partner_x * H + k * CH, CH), :],
                xstore_sems.at[k],
            )
            st.start()
            xstores.append(st)

        for k in range(NC - NSEND, NC):
            y_rdmas[k].wait_send()
        for k in range(NC):
            x_rdmas[k].wait_send()
            ystores[k].wait()
            xstores[k].wait()
        for j in (NLC - 2, NLC - 1):
            lstores[j].wait()

    return pl.pallas_call(
        body,
        out_shape=jax.ShapeDtypeStruct((2 * M, NCOL), jnp.bfloat16),
        in_specs=[pl.BlockSpec(memory_space=pl.ANY)],
        out_specs=pl.BlockSpec(memory_space=pl.ANY),
        scratch_shapes=[
            pltpu.VMEM((2, CH, NCOL), jnp.float32),
            pltpu.VMEM((NSEND, CH, NCOL), jnp.bfloat16),
            pltpu.VMEM((NC, CH, NCOL), jnp.bfloat16),
            pltpu.VMEM((NC, CH, NCOL), jnp.bfloat16),
            pltpu.VMEM((2, CHL, NCOL), jnp.float32),
            pltpu.VMEM((2, CHL, NCOL), jnp.bfloat16),
            pltpu.SemaphoreType.DMA((2,)),
            pltpu.SemaphoreType.DMA((2,)),
            pltpu.SemaphoreType.DMA((NC,)),
            pltpu.SemaphoreType.DMA((NC,)),
            pltpu.SemaphoreType.DMA((NC,)),
            pltpu.SemaphoreType.DMA((NC,)),
            pltpu.SemaphoreType.DMA((NC,)),
            pltpu.SemaphoreType.DMA((NC,)),
            pltpu.SemaphoreType.DMA((2,)),
        ],
        compiler_params=pltpu.CompilerParams(
            collective_id=0, vmem_limit_bytes=63 * 1024 * 1024
        ),
    )(x)


# device time: 237073 ns/iter; 4.7653x vs baseline; 1.0237x over previous
import jax
import jax.numpy as jnp
from jax import lax
from jax.experimental import pallas as pl
from jax.experimental.pallas import tpu as pltpu

M = 16384
N = 2048
NCOL = N // 2
H = M // 2

NC = 32
CH = H // NC
NSEND = 16

NLC = 32
CHL = M // NLC


def kernel(x):
    def body(x_ref, out_ref, ystage, sendbuf, ybuf, xbuf, lstage, lbuf,
             yld_sems, lld_sems, y_send_sems, y_recv_sems,
             x_send_sems, x_recv_sems, ystore_sems, xstore_sems, lst_sems):
        my_x = lax.axis_index("x")
        my_y = lax.axis_index("y")
        partner_y = 1 - my_y
        partner_x = 1 - my_x

        barrier_sem = pltpu.get_barrier_semaphore()
        pl.semaphore_signal(
            barrier_sem, inc=1,
            device_id=(my_x, partner_y),
            device_id_type=pl.DeviceIdType.MESH,
        )
        pl.semaphore_signal(
            barrier_sem, inc=1,
            device_id=(partner_x, my_y),
            device_id_type=pl.DeviceIdType.MESH,
        )
        pl.semaphore_wait(barrier_sem, 2)

        def yload(k):
            ld = pltpu.make_async_copy(
                x_ref.at[pl.ds(my_x * H + k * CH, CH),
                         pl.ds(partner_y * NCOL, NCOL)],
                ystage.at[k % 2],
                yld_sems.at[k % 2],
            )
            ld.start()
            return ld

        def lload(j):
            ld = pltpu.make_async_copy(
                x_ref.at[pl.ds(j * CHL, CHL), pl.ds(my_y * NCOL, NCOL)],
                lstage.at[j % 2],
                lld_sems.at[j % 2],
            )
            ld.start()
            return ld

        ylds = {0: yload(0)}
        llds = {0: lload(0)}
        y_rdmas = []
        x_rdmas = []
        ystores = []
        lstores = []

        def drain(k):
            y_rdmas[k].wait_recv()
            fwd = pltpu.make_async_remote_copy(
                src_ref=ybuf.at[k],
                dst_ref=xbuf.at[k],
                send_sem=x_send_sems.at[k],
                recv_sem=x_recv_sems.at[k],
                device_id=(partner_x, my_y),
                device_id_type=pl.DeviceIdType.MESH,
            )
            fwd.start()
            x_rdmas.append(fwd)
            st = pltpu.make_async_copy(
                ybuf.at[k],
                out_ref.at[pl.ds(partner_y * M + my_x * H + k * CH, CH), :],
                ystore_sems.at[k],
            )
            st.start()
            ystores.append(st)

        for k in range(NC):
            if k + 1 < NC:
                ylds[k + 1] = yload(k + 1)
            ylds[k].wait()
            if k >= NSEND:
                y_rdmas[k - NSEND].wait_send()
            sendbuf[k % NSEND] = ystage[k % 2].astype(jnp.bfloat16)
            rdma = pltpu.make_async_remote_copy(
                src_ref=sendbuf.at[k % NSEND],
                dst_ref=ybuf.at[k],
                send_sem=y_send_sems.at[k],
                recv_sem=y_recv_sems.at[k],
                device_id=(my_x, partner_y),
                device_id_type=pl.DeviceIdType.MESH,
            )
            rdma.start()
            y_rdmas.append(rdma)

            for j in range((NLC // NC) * k, (NLC // NC) * (k + 1)):
                if j + 1 < NLC:
                    llds[j + 1] = lload(j + 1)
                llds[j].wait()
                if j >= 2:
                    lstores[j - 2].wait()
                lbuf[j % 2] = lstage[j % 2].astype(jnp.bfloat16)
                st = pltpu.make_async_copy(
                    lbuf.at[j % 2],
                    out_ref.at[pl.ds(my_y * M + j * CHL, CHL), :],
                    lst_sems.at[j % 2],
                )
                st.start()
                lstores.append(st)

            if k >= 1:
                drain(k - 1)

        drain(NC - 1)

        xstores = []
        for k in range(NC):
            x_rdmas[k].wait_recv()
            st = pltpu.make_async_copy(
                xbuf.at[k],
                out_ref.at[pl.ds(partner_y * M + partner_x * H + k * CH, CH), :],
                xstore_sems.at[k],
            )
            st.start()
            xstores.append(st)

        for k in range(NC - NSEND, NC):
            y_rdmas[k].wait_send()
        for k in range(NC):
            x_rdmas[k].wait_send()
            ystores[k].wait()
            xstores[k].wait()
        for j in (NLC - 2, NLC - 1):
            lstores[j].wait()

    return pl.pallas_call(
        body,
        out_shape=jax.ShapeDtypeStruct((2 * M, NCOL), jnp.bfloat16),
        in_specs=[pl.BlockSpec(memory_space=pl.ANY)],
        out_specs=pl.BlockSpec(memory_space=pl.ANY),
        scratch_shapes=[
            pltpu.VMEM((2, CH, NCOL), jnp.float32),
            pltpu.VMEM((NSEND, CH, NCOL), jnp.bfloat16),
            pltpu.VMEM((NC, CH, NCOL), jnp.bfloat16),
            pltpu.VMEM((NC, CH, NCOL), jnp.bfloat16),
            pltpu.VMEM((2, CHL, NCOL), jnp.float32),
            pltpu.VMEM((2, CHL, NCOL), jnp.bfloat16),
            pltpu.SemaphoreType.DMA((2,)),
            pltpu.SemaphoreType.DMA((2,)),
            pltpu.SemaphoreType.DMA((NC,)),
            pltpu.SemaphoreType.DMA((NC,)),
            pltpu.SemaphoreType.DMA((NC,)),
            pltpu.SemaphoreType.DMA((NC,)),
            pltpu.SemaphoreType.DMA((NC,)),
            pltpu.SemaphoreType.DMA((NC,)),
            pltpu.SemaphoreType.DMA((2,)),
        ],
        compiler_params=pltpu.CompilerParams(
            collective_id=0, vmem_limit_bytes=63 * 1024 * 1024
        ),
    )(x)
